# Initial kernel scaffold; baseline (speedup 1.0000x reference)
import jax
import jax.numpy as jnp
from jax.experimental import pallas as pl


def kernel(x, edge_index, edge_attr, W1_0, b1_0, W2_0, b2_0, W1_1, b1_1, W2_1, b2_1):
    raise NotImplementedError("write your pallas kernel here")


# speedup vs baseline: 5.3907x; 5.3907x over previous
import functools

import jax
import jax.numpy as jnp
from jax import lax
from jax.experimental import pallas as pl
from jax.experimental.pallas import tpu as pltpu
from jax.experimental.pallas import tpu_sc as plsc

_N = 10000
_E = 320000
_D = 128
_CH = 128
_NW = 32
_NCHUNK = _E // _CH
_CPW = 80
_EPAD_ROWS = _NW * _CPW
_ACC_ROWS = 10112
_ZROWS = _ACC_ROWS // 16

_mesh = plsc.VectorSubcoreMesh(
    core_axis_name="c", subcore_axis_name="s", num_cores=2, num_subcores=16
)


def _sc_pass(gather: bool):
    scratch = [
        pltpu.VMEM((_CPW, _CH), jnp.int32),
        pltpu.VMEM((_CPW, _CH), jnp.int32),
        pltpu.VMEM((_CH, _D), jnp.float32),
        pltpu.VMEM_SHARED((_ACC_ROWS, _D), jnp.float32),
        pltpu.SemaphoreType.DMA,
    ]

    @functools.partial(
        pl.kernel,
        out_type=jax.ShapeDtypeStruct((2, _ACC_ROWS, _D), jnp.float32),
        mesh=_mesh,
        scratch_types=scratch,
    )
    def k(vals_hbm, src_hbm, dst_hbm, zeros_hbm, out_hbm,
          src_v, dst_v, rows_v, acc, sem):
        cid = lax.axis_index("c")
        sid = lax.axis_index("s")
        wid = cid * 16 + sid
        base = wid * _CPW

        pltpu.sync_copy(zeros_hbm.at[pl.ds(sid * _ZROWS, _ZROWS)],
                        acc.at[pl.ds(sid * _ZROWS, _ZROWS)])
        if gather:
            pltpu.sync_copy(src_hbm.at[pl.ds(base, _CPW)], src_v)
        pltpu.sync_copy(dst_hbm.at[pl.ds(base, _CPW)], dst_v)
        plsc.subcore_barrier()

        def body(j, carry):
            if gather:
                pltpu.async_copy(vals_hbm.at[src_v.at[j]], rows_v, sem).wait()
            else:
                row0 = jnp.minimum(base + j, _NCHUNK - 1) * _CH
                pltpu.sync_copy(vals_hbm.at[pl.ds(row0, _CH)], rows_v)
            pltpu.sync_copy(rows_v, acc.at[dst_v.at[j]], add=True)
            return carry

        lax.fori_loop(0, _CPW, body, 0)
        plsc.subcore_barrier()

        pltpu.sync_copy(acc.at[pl.ds(sid * _ZROWS, _ZROWS)],
                        out_hbm.at[cid, pl.ds(sid * _ZROWS, _ZROWS)])

    return k


_sc_gather_scatter = _sc_pass(gather=True)
_sc_scatter_ea = _sc_pass(gather=False)


def _mlp_body(ae_ref, y_ref, h_ref, w1a_ref, w1b_ref, w2_ref, b1_ref, b2_ref,
              o_ref, *, relu_out):
    ae = ae_ref[0] + ae_ref[1]
    y = y_ref[0] + y_ref[1] + h_ref[...]
    hid = (jnp.dot(ae, w1a_ref[...], preferred_element_type=jnp.float32)
           + jnp.dot(y, w1b_ref[...], preferred_element_type=jnp.float32)
           + b1_ref[...])
    hid = jnp.maximum(hid, 0.0)
    out = jnp.dot(hid, w2_ref[...], preferred_element_type=jnp.float32) + b2_ref[...]
    if relu_out:
        out = jnp.maximum(out, 0.0)
    o_ref[...] = out


def _mlp(ae2, y2, h, W1, b1, W2, b2, relu_out):
    B = 1000
    W1a = W1[:_D]
    W1b = W1[_D:]
    b1e = (b1 + W1[_D - 1]).reshape(1, 2 * _D)
    b2r = b2.reshape(1, _D)
    return pl.pallas_call(
        functools.partial(_mlp_body, relu_out=relu_out),
        grid=(_N // B,),
        in_specs=[
            pl.BlockSpec((2, B, _D), lambda i: (0, i, 0)),
            pl.BlockSpec((2, B, _D), lambda i: (0, i, 0)),
            pl.BlockSpec((B, _D), lambda i: (i, 0)),
            pl.BlockSpec((_D, 2 * _D), lambda i: (0, 0)),
            pl.BlockSpec((_D, 2 * _D), lambda i: (0, 0)),
            pl.BlockSpec((2 * _D, _D), lambda i: (0, 0)),
            pl.BlockSpec((1, 2 * _D), lambda i: (0, 0)),
            pl.BlockSpec((1, _D), lambda i: (0, 0)),
        ],
        out_specs=pl.BlockSpec((B, _D), lambda i: (i, 0)),
        out_shape=jax.ShapeDtypeStruct((_N, _D), jnp.float32),
    )(ae2, y2, h, W1a, W1b, W2, b1e, b2r)


def kernel(x, edge_index, edge_attr,
           W1_0, b1_0, W2_0, b2_0,
           W1_1, b1_1, W2_1, b2_1):
    src = edge_index[0]
    dst = edge_index[1]
    npad = _EPAD_ROWS * _CH - _E
    srcP = jnp.concatenate([src, jnp.zeros((npad,), jnp.int32)]).reshape(_EPAD_ROWS, _CH)
    dstP = jnp.concatenate([dst, jnp.full((npad,), _N, jnp.int32)]).reshape(_EPAD_ROWS, _CH)
    zeros_nd = jnp.zeros((_ACC_ROWS, _D), jnp.float32)

    ae2 = _sc_scatter_ea(edge_attr, srcP, dstP, zeros_nd)
    y0 = _sc_gather_scatter(x, srcP, dstP, zeros_nd)
    h1 = _mlp(ae2, y0, x, W1_0, b1_0, W2_0, b2_0, relu_out=True)
    y1 = _sc_gather_scatter(h1, srcP, dstP, zeros_nd)
    h2 = _mlp(ae2, y1, h1, W1_1, b1_1, W2_1, b2_1, relu_out=False)
    return h2


# speedup vs baseline: 6.1737x; 1.1452x over previous
import functools

import jax
import jax.numpy as jnp
from jax import lax
from jax.experimental import pallas as pl
from jax.experimental.pallas import tpu as pltpu
from jax.experimental.pallas import tpu_sc as plsc

_N = 10000
_E = 320000
_D = 128
_CH = 128
_NW = 32
_NCHUNK = _E // _CH
_CPW = 80
_EPAD_ROWS = _NW * _CPW
_ACC_ROWS = 10016
_ZOFF = 624
_ZWIN = 656
_NPH = 2
_CPP = _CPW // _NPH

_mesh = plsc.VectorSubcoreMesh(
    core_axis_name="c", subcore_axis_name="s", num_cores=2, num_subcores=16
)


def _sc_pass(gather: bool):
    nbuf = 2
    scratch = [
        pltpu.VMEM((_CPP, _CH), jnp.int32),
        pltpu.VMEM((_CPP, _CH), jnp.int32),
        [pltpu.VMEM((_CH, _D), jnp.float32) for _ in range(nbuf)],
        pltpu.VMEM_SHARED((_ACC_ROWS, _D), jnp.float32),
        [pltpu.SemaphoreType.DMA for _ in range(nbuf)],
    ]

    @functools.partial(
        pl.kernel,
        out_type=jax.ShapeDtypeStruct((2, _ACC_ROWS, _D), jnp.float32),
        mesh=_mesh,
        scratch_types=scratch,
    )
    def k(vals_hbm, src_hbm, dst_hbm, zeros_hbm, out_hbm,
          src_v, dst_v, bufs, acc, sems):
        cid = lax.axis_index("c")
        sid = lax.axis_index("s")
        wid = cid * 16 + sid
        base = wid * _CPW

        pltpu.sync_copy(zeros_hbm.at[pl.ds(sid * _ZOFF, _ZWIN)],
                        acc.at[pl.ds(sid * _ZOFF, _ZWIN)])
        plsc.subcore_barrier()

        for p in range(_NPH):
            pbase = base + p * _CPP
            if gather:
                pltpu.sync_copy(src_hbm.at[pl.ds(pbase, _CPP)], src_v)
            pltpu.sync_copy(dst_hbm.at[pl.ds(pbase, _CPP)], dst_v)

            def _fetch(j, b):
                if gather:
                    src_ref = vals_hbm.at[src_v.at[j]]
                else:
                    row0 = jnp.minimum(pbase + j, _NCHUNK - 1) * _CH
                    src_ref = vals_hbm.at[pl.ds(row0, _CH)]
                return pltpu.make_async_copy(src_ref, bufs[b], sems[b])

            for b in range(nbuf):
                _fetch(b, b).start()

            def body(jj, carry):
                for b in range(nbuf):
                    j = jj * nbuf + b
                    _fetch(j, b).wait()
                    pltpu.sync_copy(bufs[b], acc.at[dst_v.at[j]], add=True)

                    @pl.when(j + nbuf < _CPP)
                    def _():
                        _fetch(j + nbuf, b).start()
                return carry

            lax.fori_loop(0, _CPP // nbuf, body, 0)

        plsc.subcore_barrier()

        pltpu.sync_copy(acc.at[pl.ds(sid * _ZOFF, _ZWIN)],
                        out_hbm.at[cid, pl.ds(sid * _ZOFF, _ZWIN)])

    return k


_sc_gather_scatter = _sc_pass(gather=True)
_sc_scatter_ea = _sc_pass(gather=False)


def _mlp_body(ae_ref, y_ref, h_ref, w1a_ref, w1b_ref, w2_ref, b1_ref, b2_ref,
              o_ref, *, relu_out):
    ae = ae_ref[0] + ae_ref[1]
    y = y_ref[0] + y_ref[1] + h_ref[...]
    hid = (jnp.dot(ae, w1a_ref[...], preferred_element_type=jnp.float32)
           + jnp.dot(y, w1b_ref[...], preferred_element_type=jnp.float32)
           + b1_ref[...])
    hid = jnp.maximum(hid, 0.0)
    out = jnp.dot(hid, w2_ref[...], preferred_element_type=jnp.float32) + b2_ref[...]
    if relu_out:
        out = jnp.maximum(out, 0.0)
    o_ref[...] = out


def _mlp(ae2, y2, h, W1, b1, W2, b2, relu_out):
    B = 1000
    W1a = W1[:_D]
    W1b = W1[_D:]
    b1e = (b1 + W1[_D - 1]).reshape(1, 2 * _D)
    b2r = b2.reshape(1, _D)
    return pl.pallas_call(
        functools.partial(_mlp_body, relu_out=relu_out),
        grid=(_N // B,),
        in_specs=[
            pl.BlockSpec((2, B, _D), lambda i: (0, i, 0)),
            pl.BlockSpec((2, B, _D), lambda i: (0, i, 0)),
            pl.BlockSpec((B, _D), lambda i: (i, 0)),
            pl.BlockSpec((_D, 2 * _D), lambda i: (0, 0)),
            pl.BlockSpec((_D, 2 * _D), lambda i: (0, 0)),
            pl.BlockSpec((2 * _D, _D), lambda i: (0, 0)),
            pl.BlockSpec((1, 2 * _D), lambda i: (0, 0)),
            pl.BlockSpec((1, _D), lambda i: (0, 0)),
        ],
        out_specs=pl.BlockSpec((B, _D), lambda i: (i, 0)),
        out_shape=jax.ShapeDtypeStruct((_N, _D), jnp.float32),
    )(ae2, y2, h, W1a, W1b, W2, b1e, b2r)


def kernel(x, edge_index, edge_attr,
           W1_0, b1_0, W2_0, b2_0,
           W1_1, b1_1, W2_1, b2_1):
    src = edge_index[0]
    dst = edge_index[1]
    npad = _EPAD_ROWS * _CH - _E
    srcP = jnp.concatenate([src, jnp.zeros((npad,), jnp.int32)]).reshape(_EPAD_ROWS, _CH)
    dstP = jnp.concatenate([dst, jnp.full((npad,), _N, jnp.int32)]).reshape(_EPAD_ROWS, _CH)
    zeros_nd = jnp.zeros((_ACC_ROWS, _D), jnp.float32)

    ae2 = _sc_scatter_ea(edge_attr, srcP, dstP, zeros_nd)
    y0 = _sc_gather_scatter(x, srcP, dstP, zeros_nd)
    h1 = _mlp(ae2, y0, x, W1_0, b1_0, W2_0, b2_0, relu_out=True)
    y1 = _sc_gather_scatter(h1, srcP, dstP, zeros_nd)
    h2 = _mlp(ae2, y1, h1, W1_1, b1_1, W2_1, b2_1, relu_out=False)
    return h2


# speedup vs baseline: 6.1758x; 1.0003x over previous
import functools

import jax
import jax.numpy as jnp
from jax import lax
from jax.experimental import pallas as pl
from jax.experimental.pallas import tpu as pltpu
from jax.experimental.pallas import tpu_sc as plsc

_N = 10000
_E = 320000
_D = 128
_CH = 128
_NW = 32
_NCHUNK = _E // _CH
_CPW = 80
_EPAD_ROWS = _NW * _CPW
_ACC_ROWS = 10016
_ZOFF = 624
_ZWIN = 656
_NPH = 2
_CPP = _CPW // _NPH

_mesh = plsc.VectorSubcoreMesh(
    core_axis_name="c", subcore_axis_name="s", num_cores=2, num_subcores=16
)


def _sc_pass(gather: bool):
    nbuf = 2
    scratch = [
        pltpu.VMEM((_CPP, _CH), jnp.int32),
        pltpu.VMEM((_CPP, _CH), jnp.int32),
        [pltpu.VMEM((_CH, _D), jnp.float32) for _ in range(nbuf)],
        pltpu.VMEM_SHARED((_ACC_ROWS, _D), jnp.float32),
        [pltpu.SemaphoreType.DMA for _ in range(nbuf)],
    ]

    @functools.partial(
        pl.kernel,
        out_type=jax.ShapeDtypeStruct((2, _ACC_ROWS, _D), jnp.float32),
        mesh=_mesh,
        scratch_types=scratch,
    )
    def k(vals_hbm, src_hbm, dst_hbm, zeros_hbm, out_hbm,
          src_v, dst_v, bufs, acc, sems):
        cid = lax.axis_index("c")
        sid = lax.axis_index("s")
        wid = cid * 16 + sid
        base = wid * _CPW

        pltpu.sync_copy(zeros_hbm.at[pl.ds(sid * _ZOFF, _ZWIN)],
                        acc.at[pl.ds(sid * _ZOFF, _ZWIN)])
        plsc.subcore_barrier()

        for p in range(_NPH):
            pbase = base + p * _CPP
            if gather:
                pltpu.sync_copy(src_hbm.at[pl.ds(pbase, _CPP)], src_v)
            pltpu.sync_copy(dst_hbm.at[pl.ds(pbase, _CPP)], dst_v)

            subs = 4 if gather else 1
            rp = _CH // subs

            def _fire(j, b):
                if gather:
                    for t in range(subs):
                        pltpu.make_async_copy(
                            vals_hbm.at[src_v.at[j, pl.ds(t * rp, rp)]],
                            bufs[b].at[pl.ds(t * rp, rp)],
                            sems[b]).start()
                else:
                    row0 = jnp.minimum(pbase + j, _NCHUNK - 1) * _CH
                    pltpu.make_async_copy(
                        vals_hbm.at[pl.ds(row0, _CH)], bufs[b], sems[b]).start()

            def _wait(b):
                pltpu.make_async_copy(
                    vals_hbm.at[pl.ds(0, _CH)], bufs[b], sems[b]).wait()

            for b in range(nbuf):
                _fire(b, b)

            def body(jj, carry):
                for b in range(nbuf):
                    j = jj * nbuf + b
                    _wait(b)
                    pltpu.sync_copy(bufs[b], acc.at[dst_v.at[j]], add=True)

                    @pl.when(j + nbuf < _CPP)
                    def _():
                        _fire(j + nbuf, b)
                return carry

            lax.fori_loop(0, _CPP // nbuf, body, 0)

        plsc.subcore_barrier()

        pltpu.sync_copy(acc.at[pl.ds(sid * _ZOFF, _ZWIN)],
                        out_hbm.at[cid, pl.ds(sid * _ZOFF, _ZWIN)])

    return k


_sc_gather_scatter = _sc_pass(gather=True)
_sc_scatter_ea = _sc_pass(gather=False)


def _mlp_body(ae_ref, y_ref, h_ref, w1a_ref, w1b_ref, w2_ref, b1_ref, b2_ref,
              o_ref, *, relu_out):
    ae = ae_ref[0] + ae_ref[1]
    y = y_ref[0] + y_ref[1] + h_ref[...]
    hid = (jnp.dot(ae, w1a_ref[...], preferred_element_type=jnp.float32)
           + jnp.dot(y, w1b_ref[...], preferred_element_type=jnp.float32)
           + b1_ref[...])
    hid = jnp.maximum(hid, 0.0)
    out = jnp.dot(hid, w2_ref[...], preferred_element_type=jnp.float32) + b2_ref[...]
    if relu_out:
        out = jnp.maximum(out, 0.0)
    o_ref[...] = out


def _mlp(ae2, y2, h, W1, b1, W2, b2, relu_out):
    B = 1000
    W1a = W1[:_D]
    W1b = W1[_D:]
    b1e = (b1 + W1[_D - 1]).reshape(1, 2 * _D)
    b2r = b2.reshape(1, _D)
    return pl.pallas_call(
        functools.partial(_mlp_body, relu_out=relu_out),
        grid=(_N // B,),
        in_specs=[
            pl.BlockSpec((2, B, _D), lambda i: (0, i, 0)),
            pl.BlockSpec((2, B, _D), lambda i: (0, i, 0)),
            pl.BlockSpec((B, _D), lambda i: (i, 0)),
            pl.BlockSpec((_D, 2 * _D), lambda i: (0, 0)),
            pl.BlockSpec((_D, 2 * _D), lambda i: (0, 0)),
            pl.BlockSpec((2 * _D, _D), lambda i: (0, 0)),
            pl.BlockSpec((1, 2 * _D), lambda i: (0, 0)),
            pl.BlockSpec((1, _D), lambda i: (0, 0)),
        ],
        out_specs=pl.BlockSpec((B, _D), lambda i: (i, 0)),
        out_shape=jax.ShapeDtypeStruct((_N, _D), jnp.float32),
    )(ae2, y2, h, W1a, W1b, W2, b1e, b2r)


def kernel(x, edge_index, edge_attr,
           W1_0, b1_0, W2_0, b2_0,
           W1_1, b1_1, W2_1, b2_1):
    src = edge_index[0]
    dst = edge_index[1]
    npad = _EPAD_ROWS * _CH - _E
    srcP = jnp.concatenate([src, jnp.zeros((npad,), jnp.int32)]).reshape(_EPAD_ROWS, _CH)
    dstP = jnp.concatenate([dst, jnp.full((npad,), _N, jnp.int32)]).reshape(_EPAD_ROWS, _CH)
    zeros_nd = jnp.zeros((_ACC_ROWS, _D), jnp.float32)

    ae2 = _sc_scatter_ea(edge_attr, srcP, dstP, zeros_nd)
    y0 = _sc_gather_scatter(x, srcP, dstP, zeros_nd)
    h1 = _mlp(ae2, y0, x, W1_0, b1_0, W2_0, b2_0, relu_out=True)
    y1 = _sc_gather_scatter(h1, srcP, dstP, zeros_nd)
    h2 = _mlp(ae2, y1, h1, W1_1, b1_1, W2_1, b2_1, relu_out=False)
    return h2


# speedup vs baseline: 6.6308x; 1.0737x over previous
import functools

import jax
import jax.numpy as jnp
from jax import lax
from jax.experimental import pallas as pl
from jax.experimental.pallas import tpu as pltpu
from jax.experimental.pallas import tpu_sc as plsc

_N = 10000
_E = 320000
_D = 128
_CH = 128
_NW = 32
_NCHUNK = _E // _CH
_CPW = 80
_EPAD_ROWS = _NW * _CPW
_ACC_ROWS = 10016
_ZOFF = 624
_ZWIN = 656
_NPH = 2
_CPP = _CPW // _NPH

_mesh = plsc.VectorSubcoreMesh(
    core_axis_name="c", subcore_axis_name="s", num_cores=2, num_subcores=16
)


_C0 = 2048
_CPP_G = 32
_PH0 = _C0 // 16 // _CPP_G


def _sc_pass(gather: bool):
    nbuf = 2
    cpp = _CPP_G if gather else _CPP
    scratch = [
        pltpu.VMEM((cpp, _CH), jnp.int32),
        pltpu.VMEM((cpp, _CH), jnp.int32),
        [pltpu.VMEM((_CH, _D), jnp.float32) for _ in range(nbuf)],
        pltpu.VMEM_SHARED((_ACC_ROWS, _D), jnp.float32),
        [pltpu.SemaphoreType.DMA for _ in range(nbuf)],
    ]

    @functools.partial(
        pl.kernel,
        out_type=jax.ShapeDtypeStruct((2, _ACC_ROWS, _D), jnp.float32),
        mesh=_mesh,
        scratch_types=scratch,
    )
    def k(vals_hbm, src_hbm, dst_hbm, zeros_hbm, out_hbm,
          src_v, dst_v, bufs, acc, sems):
        cid = lax.axis_index("c")
        sid = lax.axis_index("s")
        wid = cid * 16 + sid

        pltpu.sync_copy(zeros_hbm.at[pl.ds(sid * _ZOFF, _ZWIN)],
                        acc.at[pl.ds(sid * _ZOFF, _ZWIN)])
        plsc.subcore_barrier()

        nphase = _PH0 if gather else _NPH

        for p in range(nphase):
            if gather:
                pbase = jnp.where(cid == 0,
                                  sid * (_C0 // 16) + p * _CPP_G,
                                  _C0 + sid * _CPP_G)
                pred = (cid == 0) | (p == 0)
            else:
                pbase = wid * _CPW + p * _CPP
                pred = jnp.bool_(True)

            @pl.when(pred)
            def _phase(pbase=pbase):
                if gather:
                    pltpu.sync_copy(src_hbm.at[pl.ds(pbase, cpp)], src_v)
                pltpu.sync_copy(dst_hbm.at[pl.ds(pbase, cpp)], dst_v)

                subs = 4 if gather else 1
                rp = _CH // subs

                def _fire(j, b):
                    if gather:
                        for t in range(subs):
                            pltpu.make_async_copy(
                                vals_hbm.at[src_v.at[j, pl.ds(t * rp, rp)]],
                                bufs[b].at[pl.ds(t * rp, rp)],
                                sems[b]).start()
                    else:
                        row0 = jnp.minimum(pbase + j, _NCHUNK - 1) * _CH
                        pltpu.make_async_copy(
                            vals_hbm.at[pl.ds(row0, _CH)], bufs[b], sems[b]).start()

                def _wait(b):
                    pltpu.make_async_copy(
                        vals_hbm.at[pl.ds(0, _CH)], bufs[b], sems[b]).wait()

                for b in range(nbuf):
                    _fire(b, b)

                def body(jj, carry):
                    for b in range(nbuf):
                        j = jj * nbuf + b
                        _wait(b)
                        pltpu.sync_copy(bufs[b], acc.at[dst_v.at[j]], add=True)

                        @pl.when(j + nbuf < cpp)
                        def _():
                            _fire(j + nbuf, b)
                    return carry

                lax.fori_loop(0, cpp // nbuf, body, 0)

        plsc.subcore_barrier()

        pltpu.sync_copy(acc.at[pl.ds(sid * _ZOFF, _ZWIN)],
                        out_hbm.at[cid, pl.ds(sid * _ZOFF, _ZWIN)])

    return k


_sc_gather_scatter = _sc_pass(gather=True)
_sc_scatter_ea = _sc_pass(gather=False)


def _mlp_body(ae_ref, y_ref, h_ref, w1a_ref, w1b_ref, w2_ref, b1_ref, b2_ref,
              o_ref, *, relu_out):
    ae = ae_ref[0] + ae_ref[1]
    y = y_ref[0] + y_ref[1] + h_ref[...]
    hid = (jnp.dot(ae, w1a_ref[...], preferred_element_type=jnp.float32)
           + jnp.dot(y, w1b_ref[...], preferred_element_type=jnp.float32)
           + b1_ref[...])
    hid = jnp.maximum(hid, 0.0)
    out = jnp.dot(hid, w2_ref[...], preferred_element_type=jnp.float32) + b2_ref[...]
    if relu_out:
        out = jnp.maximum(out, 0.0)
    o_ref[...] = out


def _mlp(ae2, y2, h, W1, b1, W2, b2, relu_out):
    B = 1000
    W1a = W1[:_D]
    W1b = W1[_D:]
    b1e = (b1 + W1[_D - 1]).reshape(1, 2 * _D)
    b2r = b2.reshape(1, _D)
    return pl.pallas_call(
        functools.partial(_mlp_body, relu_out=relu_out),
        grid=(_N // B,),
        in_specs=[
            pl.BlockSpec((2, B, _D), lambda i: (0, i, 0)),
            pl.BlockSpec((2, B, _D), lambda i: (0, i, 0)),
            pl.BlockSpec((B, _D), lambda i: (i, 0)),
            pl.BlockSpec((_D, 2 * _D), lambda i: (0, 0)),
            pl.BlockSpec((_D, 2 * _D), lambda i: (0, 0)),
            pl.BlockSpec((2 * _D, _D), lambda i: (0, 0)),
            pl.BlockSpec((1, 2 * _D), lambda i: (0, 0)),
            pl.BlockSpec((1, _D), lambda i: (0, 0)),
        ],
        out_specs=pl.BlockSpec((B, _D), lambda i: (i, 0)),
        out_shape=jax.ShapeDtypeStruct((_N, _D), jnp.float32),
    )(ae2, y2, h, W1a, W1b, W2, b1e, b2r)


def kernel(x, edge_index, edge_attr,
           W1_0, b1_0, W2_0, b2_0,
           W1_1, b1_1, W2_1, b2_1):
    src = edge_index[0]
    dst = edge_index[1]
    npad = _EPAD_ROWS * _CH - _E
    srcP = jnp.concatenate([src, jnp.zeros((npad,), jnp.int32)]).reshape(_EPAD_ROWS, _CH)
    dstP = jnp.concatenate([dst, jnp.full((npad,), _N, jnp.int32)]).reshape(_EPAD_ROWS, _CH)
    zeros_nd = jnp.zeros((_ACC_ROWS, _D), jnp.float32)

    ae2 = _sc_scatter_ea(edge_attr, srcP, dstP, zeros_nd)
    y0 = _sc_gather_scatter(x, srcP, dstP, zeros_nd)
    h1 = _mlp(ae2, y0, x, W1_0, b1_0, W2_0, b2_0, relu_out=True)
    y1 = _sc_gather_scatter(h1, srcP, dstP, zeros_nd)
    h2 = _mlp(ae2, y1, h1, W1_1, b1_1, W2_1, b2_1, relu_out=False)
    return h2


# speedup vs baseline: 17.5165x; 2.6417x over previous
import functools

import jax
import jax.numpy as jnp
from jax import lax
from jax.experimental import pallas as pl
from jax.experimental.pallas import tpu as pltpu
from jax.experimental.pallas import tpu_sc as plsc

_N = 10000
_E = 320000
_D = 128
_CH = 128
_NW = 32
_NCHUNK = _E // _CH
_CPW = 80
_EPAD_ROWS = _NW * _CPW
_ACC_ROWS = 10016
_ZOFF = 624
_ZWIN = 656
_NPH = 2
_CPP = _CPW // _NPH

_mesh = plsc.VectorSubcoreMesh(
    core_axis_name="c", subcore_axis_name="s", num_cores=2, num_subcores=16
)


def _sc_pass(gather: bool):
    nbuf = 2
    cpp = _CPP
    scratch = [
        pltpu.VMEM((cpp, _CH), jnp.int32),
        pltpu.VMEM((cpp, _CH), jnp.int32),
        [pltpu.VMEM((_CH, _D), jnp.float32) for _ in range(nbuf)],
        pltpu.VMEM_SHARED((_ACC_ROWS, _D), jnp.float32),
        [pltpu.SemaphoreType.DMA for _ in range(nbuf)],
    ]

    @functools.partial(
        pl.kernel,
        out_type=jax.ShapeDtypeStruct((2, _ACC_ROWS, _D), jnp.float32),
        mesh=_mesh,
        scratch_types=scratch,
    )
    def k(vals_hbm, src_hbm, dst_hbm, zeros_hbm, out_hbm,
          src_v, dst_v, bufs, acc, sems):
        cid = lax.axis_index("c")
        sid = lax.axis_index("s")
        wid = cid * 16 + sid

        pltpu.sync_copy(zeros_hbm.at[pl.ds(sid * _ZOFF, _ZWIN)],
                        acc.at[pl.ds(sid * _ZOFF, _ZWIN)])
        plsc.subcore_barrier()

        for p in range(_NPH):
            pbase = wid * _CPW + p * _CPP

            if True:
                if gather:
                    pltpu.sync_copy(src_hbm.at[pl.ds(pbase, cpp)], src_v)
                pltpu.sync_copy(dst_hbm.at[pl.ds(pbase, cpp)], dst_v)

                subs = 4 if gather else 1
                rp = _CH // subs

                def _fire(j, b):
                    if gather:
                        for t in range(subs):
                            pltpu.make_async_copy(
                                vals_hbm.at[src_v.at[j, pl.ds(t * rp, rp)]],
                                bufs[b].at[pl.ds(t * rp, rp)],
                                sems[b]).start()
                    else:
                        row0 = jnp.minimum(pbase + j, _NCHUNK - 1) * _CH
                        pltpu.make_async_copy(
                            vals_hbm.at[pl.ds(row0, _CH)], bufs[b], sems[b]).start()

                def _wait(b):
                    pltpu.make_async_copy(
                        vals_hbm.at[pl.ds(0, _CH)], bufs[b], sems[b]).wait()

                for b in range(nbuf):
                    _fire(b, b)

                def body(jj, carry):
                    for b in range(nbuf):
                        j = jj * nbuf + b
                        _wait(b)
                        pltpu.sync_copy(bufs[b], acc.at[dst_v.at[j]], add=True)

                        @pl.when(j + nbuf < cpp)
                        def _():
                            _fire(j + nbuf, b)
                    return carry

                lax.fori_loop(0, cpp // nbuf, body, 0)

        plsc.subcore_barrier()

        pltpu.sync_copy(acc.at[pl.ds(sid * _ZOFF, _ZWIN)],
                        out_hbm.at[cid, pl.ds(sid * _ZOFF, _ZWIN)])

    return k


_sc_gather_scatter = _sc_pass(gather=True)
_sc_scatter_ea = _sc_pass(gather=False)


def _mlp_body(ae_ref, y_ref, h_ref, w1a_ref, w1b_ref, w2_ref, b1_ref, b2_ref,
              o_ref, *, relu_out):
    ae = ae_ref[0] + ae_ref[1]
    y = y_ref[0] + y_ref[1] + h_ref[...]
    hid = (jnp.dot(ae, w1a_ref[...], preferred_element_type=jnp.float32)
           + jnp.dot(y, w1b_ref[...], preferred_element_type=jnp.float32)
           + b1_ref[...])
    hid = jnp.maximum(hid, 0.0)
    out = jnp.dot(hid, w2_ref[...], preferred_element_type=jnp.float32) + b2_ref[...]
    if relu_out:
        out = jnp.maximum(out, 0.0)
    o_ref[...] = out


def _mlp(ae2, y2, h, W1, b1, W2, b2, relu_out):
    B = 1000
    W1a = W1[:_D]
    W1b = W1[_D:]
    b1e = (b1 + W1[_D - 1]).reshape(1, 2 * _D)
    b2r = b2.reshape(1, _D)
    return pl.pallas_call(
        functools.partial(_mlp_body, relu_out=relu_out),
        grid=(_N // B,),
        in_specs=[
            pl.BlockSpec((2, B, _D), lambda i: (0, i, 0)),
            pl.BlockSpec((2, B, _D), lambda i: (0, i, 0)),
            pl.BlockSpec((B, _D), lambda i: (i, 0)),
            pl.BlockSpec((_D, 2 * _D), lambda i: (0, 0)),
            pl.BlockSpec((_D, 2 * _D), lambda i: (0, 0)),
            pl.BlockSpec((2 * _D, _D), lambda i: (0, 0)),
            pl.BlockSpec((1, 2 * _D), lambda i: (0, 0)),
            pl.BlockSpec((1, _D), lambda i: (0, 0)),
        ],
        out_specs=pl.BlockSpec((B, _D), lambda i: (i, 0)),
        out_shape=jax.ShapeDtypeStruct((_N, _D), jnp.float32),
    )(ae2, y2, h, W1a, W1b, W2, b1e, b2r)


def kernel(x, edge_index, edge_attr,
           W1_0, b1_0, W2_0, b2_0,
           W1_1, b1_1, W2_1, b2_1):
    src = edge_index[0]
    dst = edge_index[1]
    npad = _EPAD_ROWS * _CH - _E
    pad_src = jnp.arange(npad, dtype=jnp.int32) % _N
    srcP = jnp.concatenate([src, pad_src]).reshape(_EPAD_ROWS, _CH)
    dstP = jnp.concatenate([dst, jnp.full((npad,), _N, jnp.int32)]).reshape(_EPAD_ROWS, _CH)
    zeros_nd = jnp.zeros((_ACC_ROWS, _D), jnp.float32)

    ae2 = _sc_scatter_ea(edge_attr, srcP, dstP, zeros_nd)
    y0 = _sc_gather_scatter(x, srcP, dstP, zeros_nd)
    h1 = _mlp(ae2, y0, x, W1_0, b1_0, W2_0, b2_0, relu_out=True)
    y1 = _sc_gather_scatter(h1, srcP, dstP, zeros_nd)
    h2 = _mlp(ae2, y1, h1, W1_1, b1_1, W2_1, b2_1, relu_out=False)
    return h2


# speedup vs baseline: 17.7047x; 1.0107x over previous
import functools

import jax
import jax.numpy as jnp
from jax import lax
from jax.experimental import pallas as pl
from jax.experimental.pallas import tpu as pltpu
from jax.experimental.pallas import tpu_sc as plsc

_N = 10000
_E = 320000
_D = 128
_CH = 128
_NW = 32
_NCHUNK = _E // _CH
_CPW = 80
_EPAD_ROWS = _NW * _CPW
_ACC_ROWS = 10016
_ZOFF = 624
_ZWIN = 656
_NPH = 2
_CPP = _CPW // _NPH

_mesh = plsc.VectorSubcoreMesh(
    core_axis_name="c", subcore_axis_name="s", num_cores=2, num_subcores=16
)


_NBUF = 2


def _emit_pass(gather, vals_hbm, src_hbm, dst_hbm, out_hbm,
               src_v, dst_v, bufs, acc, sems, cid, sid):
    wid = cid * 16 + sid

    for p in range(_NPH):
        pbase = wid * _CPW + p * _CPP
        if gather:
            pltpu.sync_copy(src_hbm.at[pl.ds(pbase, _CPP)], src_v)
        pltpu.sync_copy(dst_hbm.at[pl.ds(pbase, _CPP)], dst_v)

        subs = 4 if gather else 1
        rp = _CH // subs

        def _fire(j, b, pbase=pbase):
            if gather:
                for t in range(subs):
                    pltpu.make_async_copy(
                        vals_hbm.at[src_v.at[j, pl.ds(t * rp, rp)]],
                        bufs[b].at[pl.ds(t * rp, rp)],
                        sems[b]).start()
            else:
                row0 = jnp.minimum(pbase + j, _NCHUNK - 1) * _CH
                pltpu.make_async_copy(
                    vals_hbm.at[pl.ds(row0, _CH)], bufs[b], sems[b]).start()

        def _wait(b):
            pltpu.make_async_copy(
                vals_hbm.at[pl.ds(0, _CH)], bufs[b], sems[b]).wait()

        for b in range(_NBUF):
            _fire(b, b)

        def body(jj, carry):
            for b in range(_NBUF):
                j = jj * _NBUF + b
                _wait(b)
                pltpu.sync_copy(bufs[b], acc.at[dst_v.at[j]], add=True)

                @pl.when(j + _NBUF < _CPP)
                def _():
                    _fire(j + _NBUF, b)
            return carry

        lax.fori_loop(0, _CPP // _NBUF, body, 0)

    plsc.subcore_barrier()
    pltpu.sync_copy(acc.at[pl.ds(sid * _ZOFF, _ZWIN)],
                    out_hbm.at[cid, pl.ds(sid * _ZOFF, _ZWIN)])


def _zero_acc(zeros_hbm, acc, sid):
    plsc.subcore_barrier()
    pltpu.sync_copy(zeros_hbm.at[pl.ds(sid * _ZOFF, _ZWIN)],
                    acc.at[pl.ds(sid * _ZOFF, _ZWIN)])
    plsc.subcore_barrier()


_scratch = [
    pltpu.VMEM((_CPP, _CH), jnp.int32),
    pltpu.VMEM((_CPP, _CH), jnp.int32),
    [pltpu.VMEM((_CH, _D), jnp.float32) for _ in range(_NBUF)],
    pltpu.VMEM_SHARED((_ACC_ROWS, _D), jnp.float32),
    [pltpu.SemaphoreType.DMA for _ in range(_NBUF)],
]

_out_t = jax.ShapeDtypeStruct((2, _ACC_ROWS, _D), jnp.float32)


@functools.partial(pl.kernel, out_type=(_out_t, _out_t), mesh=_mesh,
                   scratch_types=_scratch)
def _sc_layer0(ea_hbm, x_hbm, src_hbm, dst_hbm, zeros_hbm, ae_out, y0_out,
               src_v, dst_v, bufs, acc, sems):
    cid = lax.axis_index("c")
    sid = lax.axis_index("s")
    _zero_acc(zeros_hbm, acc, sid)
    _emit_pass(False, ea_hbm, src_hbm, dst_hbm, ae_out,
               src_v, dst_v, bufs, acc, sems, cid, sid)
    _zero_acc(zeros_hbm, acc, sid)
    _emit_pass(True, x_hbm, src_hbm, dst_hbm, y0_out,
               src_v, dst_v, bufs, acc, sems, cid, sid)


@functools.partial(pl.kernel, out_type=_out_t, mesh=_mesh,
                   scratch_types=_scratch)
def _sc_gather_scatter(vals_hbm, src_hbm, dst_hbm, zeros_hbm, out_hbm,
                       src_v, dst_v, bufs, acc, sems):
    cid = lax.axis_index("c")
    sid = lax.axis_index("s")
    _zero_acc(zeros_hbm, acc, sid)
    _emit_pass(True, vals_hbm, src_hbm, dst_hbm, out_hbm,
               src_v, dst_v, bufs, acc, sems, cid, sid)


def _mlp_body(ae_ref, y_ref, h_ref, w1a_ref, w1b_ref, w2_ref, b1_ref, b2_ref,
              o_ref, *, relu_out):
    ae = ae_ref[0] + ae_ref[1]
    y = y_ref[0] + y_ref[1] + h_ref[...]
    hid = (jnp.dot(ae, w1a_ref[...], preferred_element_type=jnp.float32)
           + jnp.dot(y, w1b_ref[...], preferred_element_type=jnp.float32)
           + b1_ref[...])
    hid = jnp.maximum(hid, 0.0)
    out = jnp.dot(hid, w2_ref[...], preferred_element_type=jnp.float32) + b2_ref[...]
    if relu_out:
        out = jnp.maximum(out, 0.0)
    o_ref[...] = out


def _mlp(ae2, y2, h, W1, b1, W2, b2, relu_out):
    B = 1000
    W1a = W1[:_D]
    W1b = W1[_D:]
    b1e = (b1 + W1[_D - 1]).reshape(1, 2 * _D)
    b2r = b2.reshape(1, _D)
    return pl.pallas_call(
        functools.partial(_mlp_body, relu_out=relu_out),
        grid=(_N // B,),
        in_specs=[
            pl.BlockSpec((2, B, _D), lambda i: (0, i, 0)),
            pl.BlockSpec((2, B, _D), lambda i: (0, i, 0)),
            pl.BlockSpec((B, _D), lambda i: (i, 0)),
            pl.BlockSpec((_D, 2 * _D), lambda i: (0, 0)),
            pl.BlockSpec((_D, 2 * _D), lambda i: (0, 0)),
            pl.BlockSpec((2 * _D, _D), lambda i: (0, 0)),
            pl.BlockSpec((1, 2 * _D), lambda i: (0, 0)),
            pl.BlockSpec((1, _D), lambda i: (0, 0)),
        ],
        out_specs=pl.BlockSpec((B, _D), lambda i: (i, 0)),
        out_shape=jax.ShapeDtypeStruct((_N, _D), jnp.float32),
    )(ae2, y2, h, W1a, W1b, W2, b1e, b2r)


def kernel(x, edge_index, edge_attr,
           W1_0, b1_0, W2_0, b2_0,
           W1_1, b1_1, W2_1, b2_1):
    src = edge_index[0]
    dst = edge_index[1]
    npad = _EPAD_ROWS * _CH - _E
    pad_src = jnp.arange(npad, dtype=jnp.int32) % _N
    srcP = jnp.concatenate([src, pad_src]).reshape(_EPAD_ROWS, _CH)
    dstP = jnp.concatenate([dst, jnp.full((npad,), _N, jnp.int32)]).reshape(_EPAD_ROWS, _CH)
    zeros_nd = jnp.zeros((_ACC_ROWS, _D), jnp.float32)

    ae2, y0 = _sc_layer0(edge_attr, x, srcP, dstP, zeros_nd)
    h1 = _mlp(ae2, y0, x, W1_0, b1_0, W2_0, b2_0, relu_out=True)
    y1 = _sc_gather_scatter(h1, srcP, dstP, zeros_nd)
    h2 = _mlp(ae2, y1, h1, W1_1, b1_1, W2_1, b2_1, relu_out=False)
    return h2


# speedup vs baseline: 17.8529x; 1.0084x over previous
import functools

import jax
import jax.numpy as jnp
from jax import lax
from jax.experimental import pallas as pl
from jax.experimental.pallas import tpu as pltpu
from jax.experimental.pallas import tpu_sc as plsc

_N = 10000
_E = 320000
_D = 128
_CH = 128
_NW = 32
_NCHUNK = _E // _CH
_CPW = 80
_EPAD_ROWS = _NW * _CPW
_ACC_ROWS = 10016
_ZOFF = 624
_ZWIN = 656
_NPH = 2
_CPP = _CPW // _NPH

_mesh = plsc.VectorSubcoreMesh(
    core_axis_name="c", subcore_axis_name="s", num_cores=2, num_subcores=16
)


_NBUF = 2


_UNROLL = 4


def _emit_pass(gather, vals_hbm, src_hbm, dst_hbm, out_hbm,
               src_v, dst_v, bufs, acc, sems, cid, sid, zero_wait=None):
    wid = cid * 16 + sid

    for p in range(_NPH):
        pbase = wid * _CPW + p * _CPP
        if gather:
            pltpu.sync_copy(src_hbm.at[pl.ds(pbase, _CPP)], src_v)
        pltpu.sync_copy(dst_hbm.at[pl.ds(pbase, _CPP)], dst_v)

        if p == 0 and zero_wait is not None:
            zero_wait()
            plsc.subcore_barrier()

        def _fire(j, b, pbase=pbase):
            if gather:
                pltpu.make_async_copy(
                    vals_hbm.at[src_v.at[j]], bufs[b], sems[b]).start()
            else:
                row0 = jnp.minimum(pbase + j, _NCHUNK - 1) * _CH
                pltpu.make_async_copy(
                    vals_hbm.at[pl.ds(row0, _CH)], bufs[b], sems[b]).start()

        def _wait(b):
            pltpu.make_async_copy(
                vals_hbm.at[pl.ds(0, _CH)], bufs[b], sems[b]).wait()

        for b in range(_NBUF):
            _fire(b, b)

        def body(jj, carry):
            for t in range(_UNROLL):
                j = jj * _UNROLL + t
                b = t % _NBUF
                _wait(b)
                pltpu.sync_copy(bufs[b], acc.at[dst_v.at[j]], add=True)

                @pl.when(j + _NBUF < _CPP)
                def _():
                    _fire(j + _NBUF, b)
            return carry

        lax.fori_loop(0, _CPP // _UNROLL, body, 0)

    plsc.subcore_barrier()
    pltpu.sync_copy(acc.at[pl.ds(sid * _ZOFF, _ZWIN)],
                    out_hbm.at[cid, pl.ds(sid * _ZOFF, _ZWIN)])


def _zero_acc_start(zeros_hbm, acc, sid, sems):
    plsc.subcore_barrier()
    cp = pltpu.make_async_copy(zeros_hbm.at[pl.ds(sid * _ZOFF, _ZWIN)],
                               acc.at[pl.ds(sid * _ZOFF, _ZWIN)], sems[0])
    cp.start()
    return cp.wait


_scratch = [
    pltpu.VMEM((_CPP, _CH), jnp.int32),
    pltpu.VMEM((_CPP, _CH), jnp.int32),
    [pltpu.VMEM((_CH, _D), jnp.float32) for _ in range(_NBUF)],
    pltpu.VMEM_SHARED((_ACC_ROWS, _D), jnp.float32),
    [pltpu.SemaphoreType.DMA for _ in range(_NBUF)],
]

_out_t = jax.ShapeDtypeStruct((2, _ACC_ROWS, _D), jnp.float32)


@functools.partial(pl.kernel, out_type=(_out_t, _out_t), mesh=_mesh,
                   scratch_types=_scratch)
def _sc_layer0(ea_hbm, x_hbm, src_hbm, dst_hbm, zeros_hbm, ae_out, y0_out,
               src_v, dst_v, bufs, acc, sems):
    cid = lax.axis_index("c")
    sid = lax.axis_index("s")
    zw = _zero_acc_start(zeros_hbm, acc, sid, sems)
    _emit_pass(False, ea_hbm, src_hbm, dst_hbm, ae_out,
               src_v, dst_v, bufs, acc, sems, cid, sid, zero_wait=zw)
    zw = _zero_acc_start(zeros_hbm, acc, sid, sems)
    _emit_pass(True, x_hbm, src_hbm, dst_hbm, y0_out,
               src_v, dst_v, bufs, acc, sems, cid, sid, zero_wait=zw)


@functools.partial(pl.kernel, out_type=_out_t, mesh=_mesh,
                   scratch_types=_scratch)
def _sc_gather_scatter(vals_hbm, src_hbm, dst_hbm, zeros_hbm, out_hbm,
                       src_v, dst_v, bufs, acc, sems):
    cid = lax.axis_index("c")
    sid = lax.axis_index("s")
    zw = _zero_acc_start(zeros_hbm, acc, sid, sems)
    _emit_pass(True, vals_hbm, src_hbm, dst_hbm, out_hbm,
               src_v, dst_v, bufs, acc, sems, cid, sid, zero_wait=zw)


def _mlp_body(ae_ref, y_ref, h_ref, w1a_ref, w1b_ref, w2_ref, b1_ref, b2_ref,
              o_ref, *, relu_out):
    ae = ae_ref[0] + ae_ref[1]
    y = y_ref[0] + y_ref[1] + h_ref[...]
    hid = (jnp.dot(ae, w1a_ref[...], preferred_element_type=jnp.float32)
           + jnp.dot(y, w1b_ref[...], preferred_element_type=jnp.float32)
           + b1_ref[...])
    hid = jnp.maximum(hid, 0.0)
    out = jnp.dot(hid, w2_ref[...], preferred_element_type=jnp.float32) + b2_ref[...]
    if relu_out:
        out = jnp.maximum(out, 0.0)
    o_ref[...] = out


def _mlp(ae2, y2, h, W1, b1, W2, b2, relu_out):
    B = 1000
    W1a = W1[:_D]
    W1b = W1[_D:]
    b1e = (b1 + W1[_D - 1]).reshape(1, 2 * _D)
    b2r = b2.reshape(1, _D)
    return pl.pallas_call(
        functools.partial(_mlp_body, relu_out=relu_out),
        grid=(_N // B,),
        in_specs=[
            pl.BlockSpec((2, B, _D), lambda i: (0, i, 0)),
            pl.BlockSpec((2, B, _D), lambda i: (0, i, 0)),
            pl.BlockSpec((B, _D), lambda i: (i, 0)),
            pl.BlockSpec((_D, 2 * _D), lambda i: (0, 0)),
            pl.BlockSpec((_D, 2 * _D), lambda i: (0, 0)),
            pl.BlockSpec((2 * _D, _D), lambda i: (0, 0)),
            pl.BlockSpec((1, 2 * _D), lambda i: (0, 0)),
            pl.BlockSpec((1, _D), lambda i: (0, 0)),
        ],
        out_specs=pl.BlockSpec((B, _D), lambda i: (i, 0)),
        out_shape=jax.ShapeDtypeStruct((_N, _D), jnp.float32),
    )(ae2, y2, h, W1a, W1b, W2, b1e, b2r)


def kernel(x, edge_index, edge_attr,
           W1_0, b1_0, W2_0, b2_0,
           W1_1, b1_1, W2_1, b2_1):
    src = edge_index[0]
    dst = edge_index[1]
    npad = _EPAD_ROWS * _CH - _E
    pad_src = jnp.arange(npad, dtype=jnp.int32) % _N
    srcP = jnp.concatenate([src, pad_src]).reshape(_EPAD_ROWS, _CH)
    dstP = jnp.concatenate([dst, jnp.full((npad,), _N, jnp.int32)]).reshape(_EPAD_ROWS, _CH)
    zeros_nd = jnp.zeros((_ACC_ROWS, _D), jnp.float32)

    ae2, y0 = _sc_layer0(edge_attr, x, srcP, dstP, zeros_nd)
    h1 = _mlp(ae2, y0, x, W1_0, b1_0, W2_0, b2_0, relu_out=True)
    y1 = _sc_gather_scatter(h1, srcP, dstP, zeros_nd)
    h2 = _mlp(ae2, y1, h1, W1_1, b1_1, W2_1, b2_1, relu_out=False)
    return h2


# speedup vs baseline: 18.0585x; 1.0115x over previous
import functools

import jax
import jax.numpy as jnp
from jax import lax
from jax.experimental import pallas as pl
from jax.experimental.pallas import tpu as pltpu
from jax.experimental.pallas import tpu_sc as plsc

_N = 10000
_E = 320000
_D = 128
_CH = 128
_NW = 32
_NCHUNK = _E // _CH
_CPW = 80
_EPAD_ROWS = _NW * _CPW
_ACC_ROWS = 10016
_ZOFF = 624
_ZWIN = 656
_NPH = 2
_CPP = _CPW // _NPH

_mesh = plsc.VectorSubcoreMesh(
    core_axis_name="c", subcore_axis_name="s", num_cores=2, num_subcores=16
)


_NBUF = 2


_UNROLL = 8


def _emit_pass(gather, vals_hbm, src_hbm, dst_hbm, out_hbm,
               src_v, dst_v, bufs, acc, sems, cid, sid, zero_wait=None):
    wid = cid * 16 + sid

    for p in range(_NPH):
        pbase = wid * _CPW + p * _CPP
        if gather:
            pltpu.sync_copy(src_hbm.at[pl.ds(pbase, _CPP)], src_v)
        pltpu.sync_copy(dst_hbm.at[pl.ds(pbase, _CPP)], dst_v)

        if p == 0 and zero_wait is not None:
            zero_wait()
            plsc.subcore_barrier()

        def _fire(j, b, pbase=pbase):
            if gather:
                pltpu.make_async_copy(
                    vals_hbm.at[src_v.at[j]], bufs[b], sems[b]).start()
            else:
                row0 = jnp.minimum(pbase + j, _NCHUNK - 1) * _CH
                pltpu.make_async_copy(
                    vals_hbm.at[pl.ds(row0, _CH)], bufs[b], sems[b]).start()

        def _wait(b):
            pltpu.make_async_copy(
                vals_hbm.at[pl.ds(0, _CH)], bufs[b], sems[b]).wait()

        for b in range(_NBUF):
            _fire(b, b)

        def body(jj, carry):
            for t in range(_UNROLL):
                j = jj * _UNROLL + t
                b = t % _NBUF
                _wait(b)
                pltpu.sync_copy(bufs[b], acc.at[dst_v.at[j]], add=True)

                @pl.when(j + _NBUF < _CPP)
                def _():
                    _fire(j + _NBUF, b)
            return carry

        lax.fori_loop(0, _CPP // _UNROLL, body, 0)

    plsc.subcore_barrier()
    pltpu.sync_copy(acc.at[pl.ds(sid * _ZOFF, _ZWIN)],
                    out_hbm.at[cid, pl.ds(sid * _ZOFF, _ZWIN)])


def _zero_acc_start(zeros_hbm, acc, sid, sems):
    plsc.subcore_barrier()
    cp = pltpu.make_async_copy(zeros_hbm.at[pl.ds(sid * _ZOFF, _ZWIN)],
                               acc.at[pl.ds(sid * _ZOFF, _ZWIN)], sems[0])
    cp.start()
    return cp.wait


_scratch = [
    pltpu.VMEM((_CPP, _CH), jnp.int32),
    pltpu.VMEM((_CPP, _CH), jnp.int32),
    [pltpu.VMEM((_CH, _D), jnp.float32) for _ in range(_NBUF)],
    pltpu.VMEM_SHARED((_ACC_ROWS, _D), jnp.float32),
    [pltpu.SemaphoreType.DMA for _ in range(_NBUF)],
]

_out_t = jax.ShapeDtypeStruct((2, _ACC_ROWS, _D), jnp.float32)


@functools.partial(pl.kernel, out_type=(_out_t, _out_t), mesh=_mesh,
                   scratch_types=_scratch)
def _sc_layer0(ea_hbm, x_hbm, src_hbm, dst_hbm, zeros_hbm, ae_out, y0_out,
               src_v, dst_v, bufs, acc, sems):
    cid = lax.axis_index("c")
    sid = lax.axis_index("s")
    zw = _zero_acc_start(zeros_hbm, acc, sid, sems)
    _emit_pass(False, ea_hbm, src_hbm, dst_hbm, ae_out,
               src_v, dst_v, bufs, acc, sems, cid, sid, zero_wait=zw)
    zw = _zero_acc_start(zeros_hbm, acc, sid, sems)
    _emit_pass(True, x_hbm, src_hbm, dst_hbm, y0_out,
               src_v, dst_v, bufs, acc, sems, cid, sid, zero_wait=zw)


@functools.partial(pl.kernel, out_type=_out_t, mesh=_mesh,
                   scratch_types=_scratch)
def _sc_gather_scatter(vals_hbm, src_hbm, dst_hbm, zeros_hbm, out_hbm,
                       src_v, dst_v, bufs, acc, sems):
    cid = lax.axis_index("c")
    sid = lax.axis_index("s")
    zw = _zero_acc_start(zeros_hbm, acc, sid, sems)
    _emit_pass(True, vals_hbm, src_hbm, dst_hbm, out_hbm,
               src_v, dst_v, bufs, acc, sems, cid, sid, zero_wait=zw)


def _mlp_body(ae_ref, y_ref, h_ref, w1a_ref, w1b_ref, w2_ref, b1_ref, b2_ref,
              o_ref, *, relu_out):
    ae = ae_ref[0] + ae_ref[1]
    y = y_ref[0] + y_ref[1] + h_ref[...]
    hid = (jnp.dot(ae, w1a_ref[...], preferred_element_type=jnp.float32)
           + jnp.dot(y, w1b_ref[...], preferred_element_type=jnp.float32)
           + b1_ref[...])
    hid = jnp.maximum(hid, 0.0)
    out = jnp.dot(hid, w2_ref[...], preferred_element_type=jnp.float32) + b2_ref[...]
    if relu_out:
        out = jnp.maximum(out, 0.0)
    o_ref[...] = out


def _mlp(ae2, y2, h, W1, b1, W2, b2, relu_out):
    B = 2000
    W1a = W1[:_D]
    W1b = W1[_D:]
    b1e = (b1 + W1[_D - 1]).reshape(1, 2 * _D)
    b2r = b2.reshape(1, _D)
    return pl.pallas_call(
        functools.partial(_mlp_body, relu_out=relu_out),
        grid=(_N // B,),
        in_specs=[
            pl.BlockSpec((2, B, _D), lambda i: (0, i, 0)),
            pl.BlockSpec((2, B, _D), lambda i: (0, i, 0)),
            pl.BlockSpec((B, _D), lambda i: (i, 0)),
            pl.BlockSpec((_D, 2 * _D), lambda i: (0, 0)),
            pl.BlockSpec((_D, 2 * _D), lambda i: (0, 0)),
            pl.BlockSpec((2 * _D, _D), lambda i: (0, 0)),
            pl.BlockSpec((1, 2 * _D), lambda i: (0, 0)),
            pl.BlockSpec((1, _D), lambda i: (0, 0)),
        ],
        out_specs=pl.BlockSpec((B, _D), lambda i: (i, 0)),
        out_shape=jax.ShapeDtypeStruct((_N, _D), jnp.float32),
    )(ae2, y2, h, W1a, W1b, W2, b1e, b2r)


def kernel(x, edge_index, edge_attr,
           W1_0, b1_0, W2_0, b2_0,
           W1_1, b1_1, W2_1, b2_1):
    src = edge_index[0]
    dst = edge_index[1]
    npad = _EPAD_ROWS * _CH - _E
    pad_src = jnp.arange(npad, dtype=jnp.int32) % _N
    srcP = jnp.concatenate([src, pad_src]).reshape(_EPAD_ROWS, _CH)
    dstP = jnp.concatenate([dst, jnp.full((npad,), _N, jnp.int32)]).reshape(_EPAD_ROWS, _CH)
    zeros_nd = jnp.zeros((_ACC_ROWS, _D), jnp.float32)

    ae2, y0 = _sc_layer0(edge_attr, x, srcP, dstP, zeros_nd)
    h1 = _mlp(ae2, y0, x, W1_0, b1_0, W2_0, b2_0, relu_out=True)
    y1 = _sc_gather_scatter(h1, srcP, dstP, zeros_nd)
    h2 = _mlp(ae2, y1, h1, W1_1, b1_1, W2_1, b2_1, relu_out=False)
    return h2
